# baseline (speedup 1.0000x reference)
import jax
import jax.numpy as jnp
from jax.experimental import pallas as pl
from jax.experimental.pallas import tpu as pltpu

B, D, H, L = 128, 1024, 1024, 2
NB = 8
BG = 4 * H // NB


def _body(xt_ref, m_ref, h0_ref, c0_ref, w_ref, b_ref,
          out_ref, nh_ref, nc_ref, gates_ref, h1s_ref):
    l = pl.program_id(0)
    j = pl.program_id(1)

    inp = jnp.where(l == 0, xt_ref[...], h1s_ref[...])
    h_prev = h0_ref[0]
    cat = jnp.concatenate(
        [inp, h_prev.astype(jnp.bfloat16)], axis=1)

    g_blk = jax.lax.dot_general(
        cat, w_ref[0], (((1,), (1,)), ((), ())),
        preferred_element_type=jnp.float32) + b_ref[0, 0, :][None, :]
    gates_ref[:, pl.ds(j * BG, BG)] = g_blk

    @pl.when(j == NB - 1)
    def _finish_layer():
        g = gates_ref[...]
        i = jax.nn.sigmoid(g[:, 0 * H:1 * H])
        f = jax.nn.sigmoid(g[:, 1 * H:2 * H])
        gg = jnp.tanh(g[:, 2 * H:3 * H])
        o = jax.nn.sigmoid(g[:, 3 * H:4 * H])
        c_new = f * c0_ref[0] + i * gg
        h_new = o * jnp.tanh(c_new)
        h1s_ref[...] = h_new.astype(jnp.bfloat16)
        m = m_ref[...] > 0
        nh_ref[0] = jnp.where(m, h_new, h0_ref[0])
        nc_ref[0] = jnp.where(m, c_new, c0_ref[0])

        @pl.when(l == 1)
        def _write_out():
            out_ref[...] = jnp.where(m, h_new, jnp.zeros_like(h_new))


@jax.jit
def kernel(x, mask, h0, c0, w_ih_l0, w_hh_l0, b_ih_l0, b_hh_l0,
           w_ih_l1, w_hh_l1, b_ih_l1, b_hh_l1):
    xt = x[:, 0, :].astype(jnp.bfloat16)
    w = jnp.stack([
        jnp.concatenate([w_ih_l0, w_hh_l0], axis=1),
        jnp.concatenate([w_ih_l1, w_hh_l1], axis=1),
    ]).astype(jnp.bfloat16)
    bias = jnp.stack([b_ih_l0 + b_hh_l0,
                      b_ih_l1 + b_hh_l1])[:, None, :]
    mf = (mask > 0).astype(jnp.float32)[:, None]

    out, new_h, new_c = pl.pallas_call(
        _body,
        grid=(L, NB),
        in_specs=[
            pl.BlockSpec((B, D), lambda l, j: (0, 0)),
            pl.BlockSpec((B, 1), lambda l, j: (0, 0)),
            pl.BlockSpec((1, B, H), lambda l, j: (l, 0, 0)),
            pl.BlockSpec((1, B, H), lambda l, j: (l, 0, 0)),
            pl.BlockSpec((1, BG, D + H), lambda l, j: (l, j, 0)),
            pl.BlockSpec((1, 1, BG), lambda l, j: (l, 0, j)),
        ],
        out_specs=[
            pl.BlockSpec((B, H), lambda l, j: (0, 0)),
            pl.BlockSpec((1, B, H), lambda l, j: (l, 0, 0)),
            pl.BlockSpec((1, B, H), lambda l, j: (l, 0, 0)),
        ],
        out_shape=[
            jax.ShapeDtypeStruct((B, H), jnp.float32),
            jax.ShapeDtypeStruct((L, B, H), jnp.float32),
            jax.ShapeDtypeStruct((L, B, H), jnp.float32),
        ],
        scratch_shapes=[
            pltpu.VMEM((B, 4 * H), jnp.float32),
            pltpu.VMEM((B, H), jnp.bfloat16),
        ],
        compiler_params=pltpu.CompilerParams(
            dimension_semantics=("arbitrary", "arbitrary"),
        ),
    )(xt, mf, h0, c0, w, bias)

    return out[:, None, :], new_h, new_c


# speedup vs baseline: 2.5658x; 2.5658x over previous
import jax
import jax.numpy as jnp
from jax.experimental import pallas as pl
from jax.experimental.pallas import tpu as pltpu

B, D, H, L = 128, 1024, 1024, 2
NB = 8
BG = 4 * H // NB


def _body(xt_ref, m_ref, h0_ref, c0_ref,
          wih0_ref, whh0_ref, wih1_ref, whh1_ref, b_ref,
          out_ref, nh_ref, nc_ref, gates_ref, h1s_ref):
    l = pl.program_id(0)
    j = pl.program_id(1)

    h_prev = h0_ref[0]

    def dots(inp, w_ih, w_hh):
        return (jax.lax.dot_general(inp, w_ih, (((1,), (1,)), ((), ())),
                                    preferred_element_type=jnp.float32)
                + jax.lax.dot_general(h_prev, w_hh, (((1,), (1,)), ((), ())),
                                      preferred_element_type=jnp.float32))

    @pl.when(l == 0)
    def _layer0():
        gates_ref[:, pl.ds(j * BG, BG)] = (
            dots(xt_ref[...], wih0_ref[...], whh0_ref[...])
            + b_ref[0, 0, :][None, :])

    @pl.when(l == 1)
    def _layer1():
        gates_ref[:, pl.ds(j * BG, BG)] = (
            dots(h1s_ref[...], wih1_ref[...], whh1_ref[...])
            + b_ref[0, 0, :][None, :])

    @pl.when(j == NB - 1)
    def _finish_layer():
        g = gates_ref[...]
        i = jax.nn.sigmoid(g[:, 0 * H:1 * H])
        f = jax.nn.sigmoid(g[:, 1 * H:2 * H])
        gg = jnp.tanh(g[:, 2 * H:3 * H])
        o = jax.nn.sigmoid(g[:, 3 * H:4 * H])
        c_new = f * c0_ref[0] + i * gg
        h_new = o * jnp.tanh(c_new)
        h1s_ref[...] = h_new
        m = m_ref[...] > 0
        nh_ref[0] = jnp.where(m, h_new, h0_ref[0])
        nc_ref[0] = jnp.where(m, c_new, c0_ref[0])

        @pl.when(l == 1)
        def _write_out():
            out_ref[...] = jnp.where(m, h_new, jnp.zeros_like(h_new))


@jax.jit
def kernel(x, mask, h0, c0, w_ih_l0, w_hh_l0, b_ih_l0, b_hh_l0,
           w_ih_l1, w_hh_l1, b_ih_l1, b_hh_l1):
    xt = x[:, 0, :]
    bias = jnp.stack([b_ih_l0 + b_hh_l0,
                      b_ih_l1 + b_hh_l1])[:, None, :]
    mf = (mask > 0).astype(jnp.float32)[:, None]

    out, new_h, new_c = pl.pallas_call(
        _body,
        grid=(L, NB),
        in_specs=[
            pl.BlockSpec((B, D), lambda l, j: (0, 0)),
            pl.BlockSpec((B, 1), lambda l, j: (0, 0)),
            pl.BlockSpec((1, B, H), lambda l, j: (l, 0, 0)),
            pl.BlockSpec((1, B, H), lambda l, j: (l, 0, 0)),
            pl.BlockSpec((BG, D), lambda l, j: (j * (1 - l) + (NB - 1) * l, 0)),
            pl.BlockSpec((BG, H), lambda l, j: (j * (1 - l) + (NB - 1) * l, 0)),
            pl.BlockSpec((BG, H), lambda l, j: (j * l, 0)),
            pl.BlockSpec((BG, H), lambda l, j: (j * l, 0)),
            pl.BlockSpec((1, 1, BG), lambda l, j: (l, 0, j)),
        ],
        out_specs=[
            pl.BlockSpec((B, H), lambda l, j: (0, 0)),
            pl.BlockSpec((1, B, H), lambda l, j: (l, 0, 0)),
            pl.BlockSpec((1, B, H), lambda l, j: (l, 0, 0)),
        ],
        out_shape=[
            jax.ShapeDtypeStruct((B, H), jnp.float32),
            jax.ShapeDtypeStruct((L, B, H), jnp.float32),
            jax.ShapeDtypeStruct((L, B, H), jnp.float32),
        ],
        scratch_shapes=[
            pltpu.VMEM((B, 4 * H), jnp.float32),
            pltpu.VMEM((B, H), jnp.float32),
        ],
        compiler_params=pltpu.CompilerParams(
            dimension_semantics=("arbitrary", "arbitrary"),
        ),
    )(xt, mf, h0, c0, w_ih_l0, w_hh_l0, w_ih_l1, w_hh_l1, bias)

    return out[:, None, :], new_h, new_c


# speedup vs baseline: 2.6822x; 1.0454x over previous
import jax
import jax.numpy as jnp
from jax.experimental import pallas as pl
from jax.experimental.pallas import tpu as pltpu

B, D, H, L = 128, 1024, 1024, 2
NB = 4
BG = 4 * H // NB


def _body(xt_ref, m_ref, h0_ref, c0_ref,
          wih0_ref, whh0_ref, wih1_ref, whh1_ref, b_ref,
          out_ref, nh_ref, nc_ref, gates_ref, h1s_ref):
    l = pl.program_id(0)
    j = pl.program_id(1)

    h_prev = h0_ref[0]

    def dots(inp, w_ih, w_hh):
        return (jax.lax.dot_general(inp, w_ih, (((1,), (1,)), ((), ())),
                                    preferred_element_type=jnp.float32)
                + jax.lax.dot_general(h_prev, w_hh, (((1,), (1,)), ((), ())),
                                      preferred_element_type=jnp.float32))

    @pl.when(l == 0)
    def _layer0():
        gates_ref[:, pl.ds(j * BG, BG)] = (
            dots(xt_ref[...], wih0_ref[...], whh0_ref[...])
            + b_ref[0, 0, :][None, :])

    @pl.when(l == 1)
    def _layer1():
        gates_ref[:, pl.ds(j * BG, BG)] = (
            dots(h1s_ref[...], wih1_ref[...], whh1_ref[...])
            + b_ref[0, 0, :][None, :])

    @pl.when(j == NB - 1)
    def _finish_layer():
        g = gates_ref[...]
        i = jax.nn.sigmoid(g[:, 0 * H:1 * H])
        f = jax.nn.sigmoid(g[:, 1 * H:2 * H])
        gg = jnp.tanh(g[:, 2 * H:3 * H])
        o = jax.nn.sigmoid(g[:, 3 * H:4 * H])
        c_new = f * c0_ref[0] + i * gg
        h_new = o * jnp.tanh(c_new)
        h1s_ref[...] = h_new
        m = m_ref[...] > 0
        nh_ref[0] = jnp.where(m, h_new, h0_ref[0])
        nc_ref[0] = jnp.where(m, c_new, c0_ref[0])

        @pl.when(l == 1)
        def _write_out():
            out_ref[...] = jnp.where(m, h_new, jnp.zeros_like(h_new))


@jax.jit
def kernel(x, mask, h0, c0, w_ih_l0, w_hh_l0, b_ih_l0, b_hh_l0,
           w_ih_l1, w_hh_l1, b_ih_l1, b_hh_l1):
    xt = x[:, 0, :]
    bias = jnp.stack([b_ih_l0 + b_hh_l0,
                      b_ih_l1 + b_hh_l1])[:, None, :]
    mf = (mask > 0).astype(jnp.float32)[:, None]

    out, new_h, new_c = pl.pallas_call(
        _body,
        grid=(L, NB),
        in_specs=[
            pl.BlockSpec((B, D), lambda l, j: (0, 0)),
            pl.BlockSpec((B, 1), lambda l, j: (0, 0)),
            pl.BlockSpec((1, B, H), lambda l, j: (l, 0, 0)),
            pl.BlockSpec((1, B, H), lambda l, j: (l, 0, 0)),
            pl.BlockSpec((BG, D), lambda l, j: (j * (1 - l) + (NB - 1) * l, 0)),
            pl.BlockSpec((BG, H), lambda l, j: (j * (1 - l) + (NB - 1) * l, 0)),
            pl.BlockSpec((BG, H), lambda l, j: (j * l, 0)),
            pl.BlockSpec((BG, H), lambda l, j: (j * l, 0)),
            pl.BlockSpec((1, 1, BG), lambda l, j: (l, 0, j)),
        ],
        out_specs=[
            pl.BlockSpec((B, H), lambda l, j: (0, 0)),
            pl.BlockSpec((1, B, H), lambda l, j: (l, 0, 0)),
            pl.BlockSpec((1, B, H), lambda l, j: (l, 0, 0)),
        ],
        out_shape=[
            jax.ShapeDtypeStruct((B, H), jnp.float32),
            jax.ShapeDtypeStruct((L, B, H), jnp.float32),
            jax.ShapeDtypeStruct((L, B, H), jnp.float32),
        ],
        scratch_shapes=[
            pltpu.VMEM((B, 4 * H), jnp.float32),
            pltpu.VMEM((B, H), jnp.float32),
        ],
        compiler_params=pltpu.CompilerParams(
            dimension_semantics=("arbitrary", "arbitrary"),
        ),
    )(xt, mf, h0, c0, w_ih_l0, w_hh_l0, w_ih_l1, w_hh_l1, bias)

    return out[:, None, :], new_h, new_c


# speedup vs baseline: 2.7623x; 1.0299x over previous
import jax
import jax.numpy as jnp
from jax.experimental import pallas as pl
from jax.experimental.pallas import tpu as pltpu

B, D, H, L = 128, 1024, 1024, 2
NB = 4
BG = 4 * H // NB


def _body(xt_ref, m_ref, h0_ref, c0_ref,
          wih0_ref, whh0_ref, wih1_ref, whh1_ref, b_ref,
          out_ref, nh_ref, nc_ref, gates_ref, h1s_ref):
    l = pl.program_id(0)
    j = pl.program_id(1)

    h_prev = h0_ref[0]

    def dots(inp, w_ih, w_hh):
        gt = (jax.lax.dot_general(w_ih, inp, (((1,), (1,)), ((), ())),
                                  preferred_element_type=jnp.float32)
              + jax.lax.dot_general(w_hh, h_prev, (((1,), (1,)), ((), ())),
                                    preferred_element_type=jnp.float32))
        return gt.T

    @pl.when(l == 0)
    def _layer0():
        gates_ref[:, pl.ds(j * BG, BG)] = (
            dots(xt_ref[...], wih0_ref[...], whh0_ref[...])
            + b_ref[0, 0, :][None, :])

    @pl.when(l == 1)
    def _layer1():
        gates_ref[:, pl.ds(j * BG, BG)] = (
            dots(h1s_ref[...], wih1_ref[...], whh1_ref[...])
            + b_ref[0, 0, :][None, :])

    @pl.when(j == NB - 1)
    def _finish_layer():
        g = gates_ref[...]
        i = jax.nn.sigmoid(g[:, 0 * H:1 * H])
        f = jax.nn.sigmoid(g[:, 1 * H:2 * H])
        gg = jnp.tanh(g[:, 2 * H:3 * H])
        o = jax.nn.sigmoid(g[:, 3 * H:4 * H])
        c_new = f * c0_ref[0] + i * gg
        h_new = o * jnp.tanh(c_new)
        h1s_ref[...] = h_new
        m = m_ref[...] > 0
        nh_ref[0] = jnp.where(m, h_new, h0_ref[0])
        nc_ref[0] = jnp.where(m, c_new, c0_ref[0])

        @pl.when(l == 1)
        def _write_out():
            out_ref[...] = jnp.where(m, h_new, jnp.zeros_like(h_new))


@jax.jit
def kernel(x, mask, h0, c0, w_ih_l0, w_hh_l0, b_ih_l0, b_hh_l0,
           w_ih_l1, w_hh_l1, b_ih_l1, b_hh_l1):
    xt = x[:, 0, :]
    bias = jnp.stack([b_ih_l0 + b_hh_l0,
                      b_ih_l1 + b_hh_l1])[:, None, :]
    mf = (mask > 0).astype(jnp.float32)[:, None]

    out, new_h, new_c = pl.pallas_call(
        _body,
        grid=(L, NB),
        in_specs=[
            pl.BlockSpec((B, D), lambda l, j: (0, 0)),
            pl.BlockSpec((B, 1), lambda l, j: (0, 0)),
            pl.BlockSpec((1, B, H), lambda l, j: (l, 0, 0)),
            pl.BlockSpec((1, B, H), lambda l, j: (l, 0, 0)),
            pl.BlockSpec((BG, D), lambda l, j: (j * (1 - l) + (NB - 1) * l, 0)),
            pl.BlockSpec((BG, H), lambda l, j: (j * (1 - l) + (NB - 1) * l, 0)),
            pl.BlockSpec((BG, H), lambda l, j: (j * l, 0)),
            pl.BlockSpec((BG, H), lambda l, j: (j * l, 0)),
            pl.BlockSpec((1, 1, BG), lambda l, j: (l, 0, j)),
        ],
        out_specs=[
            pl.BlockSpec((B, H), lambda l, j: (0, 0)),
            pl.BlockSpec((1, B, H), lambda l, j: (l, 0, 0)),
            pl.BlockSpec((1, B, H), lambda l, j: (l, 0, 0)),
        ],
        out_shape=[
            jax.ShapeDtypeStruct((B, H), jnp.float32),
            jax.ShapeDtypeStruct((L, B, H), jnp.float32),
            jax.ShapeDtypeStruct((L, B, H), jnp.float32),
        ],
        scratch_shapes=[
            pltpu.VMEM((B, 4 * H), jnp.float32),
            pltpu.VMEM((B, H), jnp.float32),
        ],
        compiler_params=pltpu.CompilerParams(
            dimension_semantics=("arbitrary", "arbitrary"),
        ),
    )(xt, mf, h0, c0, w_ih_l0, w_hh_l0, w_ih_l1, w_hh_l1, bias)

    return out[:, None, :], new_h, new_c


# speedup vs baseline: 2.8008x; 1.0139x over previous
import jax
import jax.numpy as jnp
from jax.experimental import pallas as pl
from jax.experimental.pallas import tpu as pltpu

B, D, H, L = 128, 1024, 1024, 2
NB = 4
BG = 4 * H // NB
NQ = NB // 4
BH = H // NQ


def _body(xt_ref, m_ref, h0_ref, c0_ref,
          wih0_ref, whh0_ref, wih1_ref, whh1_ref, b_ref,
          out_ref, nh_ref, nc_ref, ig_ref, fg_ref, tc_ref, h1s_ref):
    l = pl.program_id(0)
    j = pl.program_id(1)
    q = j // NQ
    col = (j % NQ) * BH

    h_prev = h0_ref[0]

    def gate_block(inp, w_ih, w_hh):
        gt = (jax.lax.dot_general(w_ih, inp, (((1,), (1,)), ((), ())),
                                  preferred_element_type=jnp.float32)
              + jax.lax.dot_general(w_hh, h_prev, (((1,), (1,)), ((), ())),
                                    preferred_element_type=jnp.float32))
        return gt.T + b_ref[0, 0, :][None, :]

    def consume(g_blk):
        cols = pl.ds(col, BH)

        @pl.when(q == 0)
        def _i_gate():
            ig_ref[:, cols] = jax.nn.sigmoid(g_blk)

        @pl.when(q == 1)
        def _f_gate():
            fg_ref[:, cols] = jax.nn.sigmoid(g_blk)

        @pl.when(q == 2)
        def _g_gate():
            c_new = (fg_ref[:, cols] * c0_ref[0, :, cols]
                     + ig_ref[:, cols] * jnp.tanh(g_blk))
            tc_ref[:, cols] = jnp.tanh(c_new)
            m = m_ref[...] > 0
            nc_ref[0, :, cols] = jnp.where(m, c_new, c0_ref[0, :, cols])

        @pl.when(q == 3)
        def _o_gate():
            h_new = jax.nn.sigmoid(g_blk) * tc_ref[:, cols]
            h1s_ref[:, cols] = h_new
            m = m_ref[...] > 0
            nh_ref[0, :, cols] = jnp.where(m, h_new, h0_ref[0, :, cols])

            @pl.when(l == 1)
            def _write_out():
                out_ref[:, cols] = jnp.where(m, h_new, jnp.zeros_like(h_new))

    @pl.when(l == 0)
    def _layer0():
        consume(gate_block(xt_ref[...], wih0_ref[...], whh0_ref[...]))

    @pl.when(l == 1)
    def _layer1():
        consume(gate_block(h1s_ref[...], wih1_ref[...], whh1_ref[...]))


@jax.jit
def kernel(x, mask, h0, c0, w_ih_l0, w_hh_l0, b_ih_l0, b_hh_l0,
           w_ih_l1, w_hh_l1, b_ih_l1, b_hh_l1):
    xt = x[:, 0, :]
    bias = jnp.stack([b_ih_l0 + b_hh_l0,
                      b_ih_l1 + b_hh_l1])[:, None, :]
    mf = (mask > 0).astype(jnp.float32)[:, None]

    out, new_h, new_c = pl.pallas_call(
        _body,
        grid=(L, NB),
        in_specs=[
            pl.BlockSpec((B, D), lambda l, j: (0, 0)),
            pl.BlockSpec((B, 1), lambda l, j: (0, 0)),
            pl.BlockSpec((1, B, H), lambda l, j: (l, 0, 0)),
            pl.BlockSpec((1, B, H), lambda l, j: (l, 0, 0)),
            pl.BlockSpec((BG, D), lambda l, j: (j * (1 - l) + (NB - 1) * l, 0)),
            pl.BlockSpec((BG, H), lambda l, j: (j * (1 - l) + (NB - 1) * l, 0)),
            pl.BlockSpec((BG, H), lambda l, j: (j * l, 0)),
            pl.BlockSpec((BG, H), lambda l, j: (j * l, 0)),
            pl.BlockSpec((1, 1, BG), lambda l, j: (l, 0, j)),
        ],
        out_specs=[
            pl.BlockSpec((B, H), lambda l, j: (0, 0)),
            pl.BlockSpec((1, B, H), lambda l, j: (l, 0, 0)),
            pl.BlockSpec((1, B, H), lambda l, j: (l, 0, 0)),
        ],
        out_shape=[
            jax.ShapeDtypeStruct((B, H), jnp.float32),
            jax.ShapeDtypeStruct((L, B, H), jnp.float32),
            jax.ShapeDtypeStruct((L, B, H), jnp.float32),
        ],
        scratch_shapes=[
            pltpu.VMEM((B, H), jnp.float32),
            pltpu.VMEM((B, H), jnp.float32),
            pltpu.VMEM((B, H), jnp.float32),
            pltpu.VMEM((B, H), jnp.float32),
        ],
        compiler_params=pltpu.CompilerParams(
            dimension_semantics=("arbitrary", "arbitrary"),
        ),
    )(xt, mf, h0, c0, w_ih_l0, w_hh_l0, w_ih_l1, w_hh_l1, bias)

    return out[:, None, :], new_h, new_c
